# baseline (speedup 1.0000x reference)
import functools

import jax
import jax.numpy as jnp
from jax import lax
from jax.experimental import pallas as pl
from jax.experimental.pallas import tpu as pltpu
from jax.experimental.pallas import tpu_sc as plsc

QUEUE_ROWS = 100000
FEAT_ROWS = 8192
DIM = 256
NUM_POP = FEAT_ROWS
QUEUE_KEEP = QUEUE_ROWS - NUM_POP
L = 16
BLOCK = 128
NBUF = 3
LAG = 1


def _build_sc_kernel():
    info = plsc.get_sparse_core_info()
    nw = info.num_cores * info.num_subcores
    per_w = FEAT_ROWS // nw
    n_blk = per_w // BLOCK

    mesh = plsc.VectorSubcoreMesh(core_axis_name="c", subcore_axis_name="s")

    @functools.partial(
        pl.kernel,
        mesh=mesh,
        out_type=jax.ShapeDtypeStruct((2 * FEAT_ROWS, DIM), jnp.float32),
        scratch_types=(
            [pltpu.VMEM((per_w,), jnp.int32)]
            + [pltpu.VMEM((BLOCK,), jnp.int32)] * (2 * n_blk)
            + [pltpu.VMEM((BLOCK,), jnp.int32)] * (2 * n_blk)
            + [pltpu.VMEM((BLOCK, DIM), jnp.float32)] * NBUF
            + [pltpu.SemaphoreType.DMA] * (2 * NBUF)
        ),
    )
    def body(queue_hbm, feat_hbm, idx_hbm, out_hbm, *scratch):
        n_steps = 2 * n_blk
        raw_v = scratch[0]
        gidx = scratch[1:1 + n_steps]
        dst = scratch[1 + n_steps:1 + 2 * n_steps]
        bufs = scratch[1 + 2 * n_steps:1 + 2 * n_steps + NBUF]
        gsem = scratch[1 + 2 * n_steps + NBUF:1 + 2 * n_steps + 2 * NBUF]
        ssem = scratch[1 + 2 * n_steps + 2 * NBUF:]

        wid = lax.axis_index("s") * info.num_cores + lax.axis_index("c")
        base = wid * per_w
        lane = lax.iota(jnp.int32, L)

        pltpu.sync_copy(idx_hbm.at[pl.ds(base, per_w)], raw_v)

        for s in range(n_steps):
            from_queue = s < n_blk
            row0 = (s % n_blk) * BLOCK
            for k in range(BLOCK // L):
                v = raw_v[pl.ds(row0 + k * L, L)]
                pos = base + row0 + k * L + lane
                if from_queue:
                    mine = v < QUEUE_KEEP
                    g = jnp.where(mine, v + NUM_POP, pos)
                else:
                    mine = v >= QUEUE_KEEP
                    g = jnp.where(mine, v - QUEUE_KEEP, pos)
                gidx[s][pl.ds(k * L, L)] = g
                dst[s][pl.ds(k * L, L)] = jnp.where(mine, pos, pos + FEAT_ROWS)

        gd = [None] * n_steps
        sd = [None] * n_steps

        def issue_scatter(t):
            gd[t].wait()
            sd[t] = pltpu.async_copy(bufs[t % NBUF], out_hbm.at[dst[t]],
                                     ssem[t % NBUF])

        for s in range(n_steps):
            b = s % NBUF
            if s >= NBUF:
                sd[s - NBUF].wait()
            table = queue_hbm if s < n_blk else feat_hbm
            gd[s] = pltpu.async_copy(table.at[gidx[s]], bufs[b], gsem[b])
            if s >= LAG:
                issue_scatter(s - LAG)
        for t in range(n_steps - LAG, n_steps):
            issue_scatter(t)
        for t in range(max(0, n_steps - NBUF), n_steps):
            sd[t].wait()

    return body


_sc_gather = _build_sc_kernel()


def kernel(queue, feat, indices):
    idx32 = indices.astype(jnp.int32)
    padded = _sc_gather(queue, feat, idx32)
    return padded[:FEAT_ROWS]


# speedup vs baseline: 1.0239x; 1.0239x over previous
import functools

import jax
import jax.numpy as jnp
from jax import lax
from jax.experimental import pallas as pl
from jax.experimental.pallas import tpu as pltpu
from jax.experimental.pallas import tpu_sc as plsc

QUEUE_ROWS = 100000
FEAT_ROWS = 8192
DIM = 256
NUM_POP = FEAT_ROWS
QUEUE_KEEP = QUEUE_ROWS - NUM_POP
L = 16
BLOCK = 64
NBUF = 6
LAG = 3


def _build_sc_kernel():
    info = plsc.get_sparse_core_info()
    nw = info.num_cores * info.num_subcores
    per_w = FEAT_ROWS // nw
    n_blk = per_w // BLOCK

    mesh = plsc.VectorSubcoreMesh(core_axis_name="c", subcore_axis_name="s")

    @functools.partial(
        pl.kernel,
        mesh=mesh,
        out_type=jax.ShapeDtypeStruct((2 * FEAT_ROWS, DIM), jnp.float32),
        scratch_types=(
            [pltpu.VMEM((per_w,), jnp.int32)]
            + [pltpu.VMEM((BLOCK,), jnp.int32)] * (2 * n_blk)
            + [pltpu.VMEM((BLOCK,), jnp.int32)] * (2 * n_blk)
            + [pltpu.VMEM((BLOCK, DIM), jnp.float32)] * NBUF
            + [pltpu.SemaphoreType.DMA] * (2 * NBUF)
        ),
    )
    def body(queue_hbm, feat_hbm, idx_hbm, out_hbm, *scratch):
        n_steps = 2 * n_blk
        raw_v = scratch[0]
        gidx = scratch[1:1 + n_steps]
        dst = scratch[1 + n_steps:1 + 2 * n_steps]
        bufs = scratch[1 + 2 * n_steps:1 + 2 * n_steps + NBUF]
        gsem = scratch[1 + 2 * n_steps + NBUF:1 + 2 * n_steps + 2 * NBUF]
        ssem = scratch[1 + 2 * n_steps + 2 * NBUF:]

        wid = lax.axis_index("s") * info.num_cores + lax.axis_index("c")
        base = wid * per_w
        lane = lax.iota(jnp.int32, L)

        pltpu.sync_copy(idx_hbm.at[pl.ds(base, per_w)], raw_v)

        for s in range(n_steps):
            from_queue = s < n_blk
            row0 = (s % n_blk) * BLOCK
            for k in range(BLOCK // L):
                v = raw_v[pl.ds(row0 + k * L, L)]
                pos = base + row0 + k * L + lane
                if from_queue:
                    mine = v < QUEUE_KEEP
                    g = jnp.where(mine, v + NUM_POP, pos)
                else:
                    mine = v >= QUEUE_KEEP
                    g = jnp.where(mine, v - QUEUE_KEEP, pos)
                gidx[s][pl.ds(k * L, L)] = g
                dst[s][pl.ds(k * L, L)] = jnp.where(mine, pos, pos + FEAT_ROWS)

        gd = [None] * n_steps
        sd = [None] * n_steps

        def issue_scatter(t):
            gd[t].wait()
            sd[t] = pltpu.async_copy(bufs[t % NBUF], out_hbm.at[dst[t]],
                                     ssem[t % NBUF])

        for s in range(n_steps):
            b = s % NBUF
            if s >= NBUF:
                sd[s - NBUF].wait()
            table = queue_hbm if s < n_blk else feat_hbm
            gd[s] = pltpu.async_copy(table.at[gidx[s]], bufs[b], gsem[b])
            if s >= LAG:
                issue_scatter(s - LAG)
        for t in range(n_steps - LAG, n_steps):
            issue_scatter(t)
        for t in range(max(0, n_steps - NBUF), n_steps):
            sd[t].wait()

    return body


_sc_gather = _build_sc_kernel()


def kernel(queue, feat, indices):
    idx32 = indices.astype(jnp.int32)
    padded = _sc_gather(queue, feat, idx32)
    return padded[:FEAT_ROWS]


# speedup vs baseline: 1.0361x; 1.0119x over previous
import functools

import jax
import jax.numpy as jnp
from jax import lax
from jax.experimental import pallas as pl
from jax.experimental.pallas import tpu as pltpu
from jax.experimental.pallas import tpu_sc as plsc

QUEUE_ROWS = 100000
FEAT_ROWS = 8192
DIM = 256
NUM_POP = FEAT_ROWS
QUEUE_KEEP = QUEUE_ROWS - NUM_POP
L = 16
BLOCK = 64
NBUF = 7
LAG = 4


def _build_sc_kernel():
    info = plsc.get_sparse_core_info()
    nw = info.num_cores * info.num_subcores
    per_w = FEAT_ROWS // nw
    n_blk = per_w // BLOCK

    mesh = plsc.VectorSubcoreMesh(core_axis_name="c", subcore_axis_name="s")

    @functools.partial(
        pl.kernel,
        mesh=mesh,
        out_type=jax.ShapeDtypeStruct((2 * FEAT_ROWS, DIM), jnp.float32),
        scratch_types=(
            [pltpu.VMEM((per_w,), jnp.int32)]
            + [pltpu.VMEM((BLOCK,), jnp.int32)] * (2 * n_blk)
            + [pltpu.VMEM((BLOCK,), jnp.int32)] * (2 * n_blk)
            + [pltpu.VMEM((BLOCK, DIM), jnp.float32)] * NBUF
            + [pltpu.SemaphoreType.DMA] * (2 * NBUF)
        ),
    )
    def body(queue_hbm, feat_hbm, idx_hbm, out_hbm, *scratch):
        n_steps = 2 * n_blk
        raw_v = scratch[0]
        gidx = scratch[1:1 + n_steps]
        dst = scratch[1 + n_steps:1 + 2 * n_steps]
        bufs = scratch[1 + 2 * n_steps:1 + 2 * n_steps + NBUF]
        gsem = scratch[1 + 2 * n_steps + NBUF:1 + 2 * n_steps + 2 * NBUF]
        ssem = scratch[1 + 2 * n_steps + 2 * NBUF:]

        wid = lax.axis_index("s") * info.num_cores + lax.axis_index("c")
        base = wid * per_w
        lane = lax.iota(jnp.int32, L)

        pltpu.sync_copy(idx_hbm.at[pl.ds(base, per_w)], raw_v)

        for s in range(n_steps):
            from_queue = s < n_blk
            row0 = (s % n_blk) * BLOCK
            for k in range(BLOCK // L):
                v = raw_v[pl.ds(row0 + k * L, L)]
                pos = base + row0 + k * L + lane
                if from_queue:
                    mine = v < QUEUE_KEEP
                    g = jnp.where(mine, v + NUM_POP, pos)
                else:
                    mine = v >= QUEUE_KEEP
                    g = jnp.where(mine, v - QUEUE_KEEP, pos)
                gidx[s][pl.ds(k * L, L)] = g
                dst[s][pl.ds(k * L, L)] = jnp.where(mine, pos, pos + FEAT_ROWS)

        gd = [None] * n_steps
        sd = [None] * n_steps

        def issue_scatter(t):
            gd[t].wait()
            sd[t] = pltpu.async_copy(bufs[t % NBUF], out_hbm.at[dst[t]],
                                     ssem[t % NBUF])

        for s in range(n_steps):
            b = s % NBUF
            if s >= NBUF:
                sd[s - NBUF].wait()
            table = queue_hbm if s < n_blk else feat_hbm
            gd[s] = pltpu.async_copy(table.at[gidx[s]], bufs[b], gsem[b])
            if s >= LAG:
                issue_scatter(s - LAG)
        for t in range(n_steps - LAG, n_steps):
            issue_scatter(t)
        for t in range(max(0, n_steps - NBUF), n_steps):
            sd[t].wait()

    return body


_sc_gather = _build_sc_kernel()


def kernel(queue, feat, indices):
    idx32 = indices.astype(jnp.int32)
    padded = _sc_gather(queue, feat, idx32)
    return padded[:FEAT_ROWS]
